# Initial kernel scaffold; baseline (speedup 1.0000x reference)
import functools

import jax
import jax.numpy as jnp
from jax import lax
from jax.experimental import pallas as pl
from jax.experimental.pallas import tpu as pltpu
from jax.experimental.pallas import tpu_sc as plsc

N_REAL = 10000
NT = 10496
NB = NT // 128
H = 64
E = 320000
NW = 32
CH = 80
EW = CH * 128
EPAD = NW * EW
ZBASE = N_REAL + 240
ZSPREAD = 255
KS = (5000, 2500, 1250)

_SC_MESH = plsc.VectorSubcoreMesh(
    core_axis_name="c", subcore_axis_name="s", num_cores=2, num_subcores=16)


def _sc_pass_a(alive, src3, dst3, zeros1d):
    @functools.partial(
        pl.kernel,
        out_type=(jax.ShapeDtypeStruct((NW, NT), jnp.float32),
                  jax.ShapeDtypeStruct((NW, CH, 128), jnp.int32)),
        mesh=_SC_MESH,
        scratch_types=[pltpu.VMEM((NT,), jnp.float32),
                       pltpu.VMEM((CH, 128), jnp.int32),
                       pltpu.VMEM((CH, 128), jnp.int32),
                       pltpu.VMEM((CH, 128), jnp.int32),
                       pltpu.VMEM((NT,), jnp.float32)],
    )
    def k(alive_hbm, src_hbm, dst_hbm, z_hbm, degp_hbm, src2_hbm,
          alive_v, src_v, dst_v, src2_v, degp_v):
        w = lax.axis_index("s") * 2 + lax.axis_index("c")
        pltpu.sync_copy(alive_hbm, alive_v)
        pltpu.sync_copy(src_hbm.at[w], src_v)
        pltpu.sync_copy(dst_hbm.at[w], dst_v)
        pltpu.sync_copy(z_hbm, degp_v)
        iota = lax.iota(jnp.int32, 16)

        @pl.loop(0, CH)
        def _(c):
            @pl.loop(0, 8)
            def _(j):
                sv = src_v[c, pl.ds(j * 16, 16)]
                dv = dst_v[c, pl.ds(j * 16, 16)]
                a_s = plsc.load_gather(alive_v, [sv])
                a_d = plsc.load_gather(alive_v, [dv])
                m = a_s * a_d
                spread = jnp.bitwise_and(iota + (c * 8 + j) * 16, ZSPREAD)
                src2_v[c, pl.ds(j * 16, 16)] = jnp.where(m > 0.0, sv,
                                                         ZBASE + spread)
                plsc.addupdate_scatter(degp_v, [dv], m)

        pltpu.sync_copy(degp_v, degp_hbm.at[w])
        pltpu.sync_copy(src2_v, src2_hbm.at[w])

    return k(alive, src3, dst3, zeros1d)


def _sc_pass_b(z, src23, dst3, zeros2d):
    @functools.partial(
        pl.kernel,
        out_type=jax.ShapeDtypeStruct((2, NT, H), jnp.float32),
        mesh=_SC_MESH,
        scratch_types=[pltpu.VMEM((CH, 128), jnp.int32),
                       pltpu.VMEM((CH, 128), jnp.int32),
                       pltpu.VMEM((128, H), jnp.float32),
                       pltpu.VMEM_SHARED((NT, H), jnp.float32)],
    )
    def k(z_hbm, src2_hbm, dst_hbm, zero_hbm, out_hbm,
          src2_v, dst_v, rows_v, table):
        cid = lax.axis_index("c")
        sid = lax.axis_index("s")
        w = sid * 2 + cid
        rows_per = NT // 16
        sl = pl.ds(sid * rows_per, rows_per)
        pltpu.sync_copy(zero_hbm.at[sl], table.at[sl])
        pltpu.sync_copy(src2_hbm.at[w], src2_v)
        pltpu.sync_copy(dst_hbm.at[w], dst_v)
        plsc.subcore_barrier()

        @pl.loop(0, CH)
        def _(c):
            pltpu.sync_copy(z_hbm.at[src2_v.at[c]], rows_v)
            pltpu.sync_copy(rows_v, table.at[dst_v.at[c]], add=True)

        plsc.subcore_barrier()
        pltpu.sync_copy(table.at[sl], out_hbm.at[cid, sl])

    return k(z, src23, dst3, zeros2d)


def _sc_pass_c(stilde, src23, dst3, zeros1d):
    @functools.partial(
        pl.kernel,
        out_type=jax.ShapeDtypeStruct((NW, NT), jnp.float32),
        mesh=_SC_MESH,
        scratch_types=[pltpu.VMEM((NT,), jnp.float32),
                       pltpu.VMEM((CH, 128), jnp.int32),
                       pltpu.VMEM((CH, 128), jnp.int32),
                       pltpu.VMEM((NT,), jnp.float32)],
    )
    def k(st_hbm, src2_hbm, dst_hbm, z_hbm, scorep_hbm,
          st_v, src2_v, dst_v, acc_v):
        w = lax.axis_index("s") * 2 + lax.axis_index("c")
        pltpu.sync_copy(st_hbm, st_v)
        pltpu.sync_copy(src2_hbm.at[w], src2_v)
        pltpu.sync_copy(dst_hbm.at[w], dst_v)
        pltpu.sync_copy(z_hbm, acc_v)

        @pl.loop(0, CH)
        def _(c):
            @pl.loop(0, 8)
            def _(j):
                sv = src2_v[c, pl.ds(j * 16, 16)]
                dv = dst_v[c, pl.ds(j * 16, 16)]
                s = plsc.load_gather(st_v, [sv])
                plsc.addupdate_scatter(acc_v, [dv], s)

        pltpu.sync_copy(acc_v, scorep_hbm.at[w])

    return k(stilde, src23, dst3, zeros1d)


def _tc_prep(x, W, degp):
    din = x.shape[1]

    def body(x_ref, w_ref, degp_ref, z_ref, xwd2_ref, dinv_ref):
        deg = jnp.sum(degp_ref[...], axis=0) + 1.0
        dinv = lax.rsqrt(deg)
        xw = jnp.dot(x_ref[...], w_ref[...],
                     preferred_element_type=jnp.float32)
        dc = dinv[:, None]
        z_ref[...] = xw * dc
        xwd2_ref[...] = xw * (dc * dc)
        dinv_ref[...] = dinv[None, :]

    return pl.pallas_call(
        body,
        grid=(NB,),
        in_specs=[pl.BlockSpec((128, din), lambda i: (i, 0)),
                  pl.BlockSpec((din, H), lambda i: (0, 0)),
                  pl.BlockSpec((NW, 128), lambda i: (0, i))],
        out_specs=[pl.BlockSpec((128, H), lambda i: (i, 0)),
                   pl.BlockSpec((128, H), lambda i: (i, 0)),
                   pl.BlockSpec((1, 128), lambda i: (i, 0))],
        out_shape=[jax.ShapeDtypeStruct((NT, H), jnp.float32),
                   jax.ShapeDtypeStruct((NT, H), jnp.float32),
                   jax.ShapeDtypeStruct((NB, 128), jnp.float32)],
    )(x, W, degp)


def _tc_post_agg(p, xwd2, dinv2, b, wsT, bs):
    def body(p_ref, xwd2_ref, dinv_ref, b_ref, ws_ref, bs_ref,
             y_ref, st_ref, ss_ref):
        i = pl.program_id(0)
        dinv = dinv_ref[0, :]
        dc = dinv[:, None]
        es = p_ref[0] + p_ref[1]
        y = jnp.maximum(es * dc + xwd2_ref[...] + b_ref[...], 0.0)
        rowid = i * 128 + lax.broadcasted_iota(jnp.int32, (128, 1), 0)
        y = jnp.where(rowid < N_REAL, y, 0.0)
        y_ref[...] = y
        shat = jnp.sum(y * ws_ref[...], axis=1)
        st_ref[...] = (shat * dinv)[None, :]
        ss_ref[...] = (shat * dinv * dinv + bs_ref[0, 0])[None, :]

    return pl.pallas_call(
        body,
        grid=(NB,),
        in_specs=[pl.BlockSpec((2, 128, H), lambda i: (0, i, 0)),
                  pl.BlockSpec((128, H), lambda i: (i, 0)),
                  pl.BlockSpec((1, 128), lambda i: (i, 0)),
                  pl.BlockSpec((1, H), lambda i: (0, 0)),
                  pl.BlockSpec((1, H), lambda i: (0, 0)),
                  pl.BlockSpec((1, 1), lambda i: (0, 0))],
        out_specs=[pl.BlockSpec((128, H), lambda i: (i, 0)),
                   pl.BlockSpec((1, 128), lambda i: (i, 0)),
                   pl.BlockSpec((1, 128), lambda i: (i, 0))],
        out_shape=[jax.ShapeDtypeStruct((NT, H), jnp.float32),
                   jax.ShapeDtypeStruct((NB, 128), jnp.float32),
                   jax.ShapeDtypeStruct((NB, 128), jnp.float32)],
    )(p, xwd2, dinv2, b, wsT, bs)


def _tc_topk(scorep3, dinv2, ss2, alive2, k):
    def body(sp_ref, dinv_ref, ss_ref, alive_ref, scale_ref, anew_ref):
        es = jnp.sum(sp_ref[...], axis=0)
        score = dinv_ref[...] * es + ss_ref[...]
        u = lax.bitcast_convert_type(score, jnp.uint32)
        flip = jnp.where(u >= jnp.uint32(0x80000000),
                         jnp.uint32(0xFFFFFFFF), jnp.uint32(0x80000000))
        key = jnp.where(alive_ref[...] > 0.0, u ^ flip, jnp.uint32(0))

        def step(i, t):
            bit = jnp.uint32(31) - i.astype(jnp.uint32)
            cand = t | (jnp.uint32(1) << bit)
            cnt = jnp.sum((key >= cand).astype(jnp.int32))
            return jnp.where(cnt >= k, cand, t)

        thr = lax.fori_loop(0, 32, step, jnp.uint32(0))
        anew = (key >= thr).astype(jnp.float32)
        anew_ref[...] = anew
        scale_ref[...] = jnp.tanh(score) * anew

    return pl.pallas_call(
        body,
        in_specs=[pl.BlockSpec((NW, NB, 128), lambda: (0, 0, 0)),
                  pl.BlockSpec((NB, 128), lambda: (0, 0)),
                  pl.BlockSpec((NB, 128), lambda: (0, 0)),
                  pl.BlockSpec((NB, 128), lambda: (0, 0))],
        out_specs=[pl.BlockSpec((NB, 128), lambda: (0, 0)),
                   pl.BlockSpec((NB, 128), lambda: (0, 0))],
        out_shape=[jax.ShapeDtypeStruct((NB, 128), jnp.float32),
                   jax.ShapeDtypeStruct((NB, 128), jnp.float32)],
    )(scorep3, dinv2, ss2, alive2)


def _tc_pool(y, scale2, alive2):
    def body(y_ref, sc_ref, a_ref, xn_ref, rp_ref):
        s = sc_ref[0, :][:, None]
        a = a_ref[0, :][:, None]
        xn = y_ref[...] * s
        xn_ref[...] = xn
        mx = jnp.max(jnp.where(a > 0.0, xn, -jnp.inf), axis=0)
        sm = jnp.sum(xn, axis=0)
        rp_ref[...] = jnp.concatenate([mx, sm])[None, :]

    return pl.pallas_call(
        body,
        grid=(NB,),
        in_specs=[pl.BlockSpec((128, H), lambda i: (i, 0)),
                  pl.BlockSpec((1, 128), lambda i: (i, 0)),
                  pl.BlockSpec((1, 128), lambda i: (i, 0))],
        out_specs=[pl.BlockSpec((128, H), lambda i: (i, 0)),
                   pl.BlockSpec((1, 128), lambda i: (i, 0))],
        out_shape=[jax.ShapeDtypeStruct((NT, H), jnp.float32),
                   jax.ShapeDtypeStruct((NB, 128), jnp.float32)],
    )(y, scale2, alive2)


def _tc_final(rp1, rp2, rp3, w1, b1, w2, b2, w3, b3):
    def body(r1, r2, r3, w1r, b1r, w2r, b2r, w3r, b3r, hn_ref, hc_ref):
        h = jnp.zeros((1, 128), jnp.float32)
        for rp, k in ((r1, KS[0]), (r2, KS[1]), (r3, KS[2])):
            v = rp[...]
            mx = jnp.max(v[:, :H], axis=0)
            sm = jnp.sum(v[:, H:], axis=0) * (1.0 / k)
            h = h + jnp.concatenate([mx, sm])[None, :]
        h = jnp.maximum(jnp.dot(h, w1r[...],
                                preferred_element_type=jnp.float32)
                        + b1r[...], 0.0)
        h = jnp.maximum(jnp.dot(h, w2r[...],
                                preferred_element_type=jnp.float32)
                        + b2r[...], 0.0)
        nrm = jnp.maximum(jnp.sqrt(jnp.sum(h * h)), 1e-12)
        hn_ref[...] = h / nrm
        hc_ref[...] = jnp.dot(h, w3r[...],
                              preferred_element_type=jnp.float32) + b3r[...]

    def full(s):
        return pl.BlockSpec(s, lambda *_: tuple(0 for _ in s))

    return pl.pallas_call(
        body,
        in_specs=[full((NB, 128)), full((NB, 128)), full((NB, 128)),
                  full((2 * H, H)), full((1, H)),
                  full((H, 32)), full((1, 32)),
                  full((32, 2)), full((1, 2))],
        out_specs=[full((1, 32)), full((1, 2))],
        out_shape=[jax.ShapeDtypeStruct((1, 32), jnp.float32),
                   jax.ShapeDtypeStruct((1, 2), jnp.float32)],
    )(rp1, rp2, rp3, w1, b1, w2, b2, w3, b3)


def _tower(x, src, dst, stages, head, consts):
    zeros1d, zeros2d, alive0 = consts
    xp = jnp.pad(x, ((0, NT - N_REAL), (0, 0)))
    pad_n = EPAD - E
    ar = jnp.arange(pad_n, dtype=jnp.int32)
    src3 = jnp.concatenate([src, ZBASE + (ar % 256)]).reshape(NW, CH, 128)
    dst3 = jnp.concatenate([dst, ar % N_REAL]).reshape(NW, CH, 128)

    alive = alive0
    xcur = xp
    rps = []
    for (W, b, wsT, bs), k in zip(stages, KS):
        degp, src2 = _sc_pass_a(alive.reshape(NT), src3, dst3, zeros1d)
        z, xwd2, dinv2 = _tc_prep(xcur, W, degp)
        p = _sc_pass_b(z, src2, dst3, zeros2d)
        y, st2, ss2 = _tc_post_agg(p, xwd2, dinv2, b, wsT, bs)
        scorep = _sc_pass_c(st2.reshape(NT), src2, dst3, zeros1d)
        scale2, anew2 = _tc_topk(scorep.reshape(NW, NB, 128), dinv2, ss2,
                                 alive, k)
        xcur, rp = _tc_pool(y, scale2, anew2)
        alive = anew2
        rps.append(rp)
    return _tc_final(rps[0], rps[1], rps[2], *head)


def kernel(x1, x2, edge_index1, edge_index2, batch1, batch2,
           W1, b1, Ws1, bs1, W2, b2, Ws2, bs2, W3, b3, Ws3, bs3,
           Wl1, bl1, Wl2, bl2, Wl3, bl3):
    zeros1d = jnp.zeros((NT,), jnp.float32)
    zeros2d = jnp.zeros((NT, H), jnp.float32)
    alive0 = (jnp.arange(NT) < N_REAL).astype(jnp.float32).reshape(NB, 128)
    consts = (zeros1d, zeros2d, alive0)

    stages = [(W1, b1.reshape(1, H), Ws1.reshape(1, H), bs1.reshape(1, 1)),
              (W2, b2.reshape(1, H), Ws2.reshape(1, H), bs2.reshape(1, 1)),
              (W3, b3.reshape(1, H), Ws3.reshape(1, H), bs3.reshape(1, 1))]
    head = (Wl1, bl1.reshape(1, H), Wl2, bl2.reshape(1, 32),
            Wl3, bl3.reshape(1, 2))

    n1, c1 = _tower(x1, edge_index1[0], edge_index1[1], stages, head, consts)
    n2, c2 = _tower(x2, edge_index2[0], edge_index2[1], stages, head, consts)
    return (n1, c1, n2, c2)


# speedup vs baseline: 42.0645x; 42.0645x over previous
import functools

import jax
import jax.numpy as jnp
from jax import lax
from jax.experimental import pallas as pl
from jax.experimental.pallas import tpu as pltpu
from jax.experimental.pallas import tpu_sc as plsc

N_REAL = 10000
NT = 10496
NB = NT // 128
H = 64
E = 320000
NW = 32
CH = 80
EW = CH * 128
EPAD = NW * EW
ZBASE = N_REAL + 240
ZSPREAD = 255
KS = (5000, 2500, 1250)

_SC_CP = pltpu.CompilerParams(needs_layout_passes=False,
                              use_tc_tiling_on_sc=False)


@functools.cache
def _sc_mesh():
    return plsc.VectorSubcoreMesh(
        core_axis_name="c", subcore_axis_name="s",
        num_cores=2, num_subcores=16)


def _sc_pass_a(alive, src3, dst3, zeros1d):
    @functools.partial(
        pl.kernel,
        out_type=(jax.ShapeDtypeStruct((NW, NT), jnp.float32),
                  jax.ShapeDtypeStruct((NW, CH, 128), jnp.int32)),
        mesh=_sc_mesh(),
        compiler_params=_SC_CP,
        scratch_types=[pltpu.VMEM((NT,), jnp.float32),
                       pltpu.VMEM((CH, 128), jnp.int32),
                       pltpu.VMEM((CH, 128), jnp.int32),
                       pltpu.VMEM((CH, 128), jnp.int32),
                       pltpu.VMEM((NT,), jnp.float32)],
    )
    def k(alive_hbm, src_hbm, dst_hbm, z_hbm, degp_hbm, src2_hbm,
          alive_v, src_v, dst_v, src2_v, degp_v):
        w = lax.axis_index("s") * 2 + lax.axis_index("c")
        pltpu.sync_copy(alive_hbm, alive_v)
        pltpu.sync_copy(src_hbm.at[w], src_v)
        pltpu.sync_copy(dst_hbm.at[w], dst_v)
        pltpu.sync_copy(z_hbm, degp_v)
        iota = lax.iota(jnp.int32, 16)

        @pl.loop(0, CH)
        def _(c):
            @pl.loop(0, 8)
            def _(j):
                sv = src_v[c, pl.ds(j * 16, 16)]
                dv = dst_v[c, pl.ds(j * 16, 16)]
                a_s = plsc.load_gather(alive_v, [sv])
                a_d = plsc.load_gather(alive_v, [dv])
                m = a_s * a_d
                spread = jnp.bitwise_and(iota + (c * 8 + j) * 16, ZSPREAD)
                src2_v[c, pl.ds(j * 16, 16)] = jnp.where(m > 0.0, sv,
                                                         ZBASE + spread)
                plsc.addupdate_scatter(degp_v, [dv], m)

        pltpu.sync_copy(degp_v, degp_hbm.at[w])
        pltpu.sync_copy(src2_v, src2_hbm.at[w])

    return k(alive, src3, dst3, zeros1d)


def _sc_pass_b(z, src23, dst3, zeros2d):
    @functools.partial(
        pl.kernel,
        out_type=jax.ShapeDtypeStruct((2, NT, H), jnp.float32),
        mesh=_sc_mesh(),
        compiler_params=_SC_CP,
        scratch_types=[pltpu.VMEM((CH, 128), jnp.int32),
                       pltpu.VMEM((CH, 128), jnp.int32),
                       pltpu.VMEM((128, H), jnp.float32),
                       pltpu.VMEM_SHARED((NT, H), jnp.float32)],
    )
    def k(z_hbm, src2_hbm, dst_hbm, zero_hbm, out_hbm,
          src2_v, dst_v, rows_v, table):
        cid = lax.axis_index("c")
        sid = lax.axis_index("s")
        w = sid * 2 + cid
        rows_per = NT // 16
        sl = pl.ds(sid * rows_per, rows_per)
        pltpu.sync_copy(zero_hbm.at[sl], table.at[sl])
        pltpu.sync_copy(src2_hbm.at[w], src2_v)
        pltpu.sync_copy(dst_hbm.at[w], dst_v)
        plsc.subcore_barrier()

        @pl.loop(0, CH)
        def _(c):
            pltpu.sync_copy(z_hbm.at[src2_v.at[c]], rows_v)
            pltpu.sync_copy(rows_v, table.at[dst_v.at[c]], add=True)

        plsc.subcore_barrier()
        pltpu.sync_copy(table.at[sl], out_hbm.at[cid, sl])

    return k(z, src23, dst3, zeros2d)


def _sc_pass_c(stilde, src23, dst3, zeros1d):
    @functools.partial(
        pl.kernel,
        out_type=jax.ShapeDtypeStruct((NW, NT), jnp.float32),
        mesh=_sc_mesh(),
        compiler_params=_SC_CP,
        scratch_types=[pltpu.VMEM((NT,), jnp.float32),
                       pltpu.VMEM((CH, 128), jnp.int32),
                       pltpu.VMEM((CH, 128), jnp.int32),
                       pltpu.VMEM((NT,), jnp.float32)],
    )
    def k(st_hbm, src2_hbm, dst_hbm, z_hbm, scorep_hbm,
          st_v, src2_v, dst_v, acc_v):
        w = lax.axis_index("s") * 2 + lax.axis_index("c")
        pltpu.sync_copy(st_hbm, st_v)
        pltpu.sync_copy(src2_hbm.at[w], src2_v)
        pltpu.sync_copy(dst_hbm.at[w], dst_v)
        pltpu.sync_copy(z_hbm, acc_v)

        @pl.loop(0, CH)
        def _(c):
            @pl.loop(0, 8)
            def _(j):
                sv = src2_v[c, pl.ds(j * 16, 16)]
                dv = dst_v[c, pl.ds(j * 16, 16)]
                s = plsc.load_gather(st_v, [sv])
                plsc.addupdate_scatter(acc_v, [dv], s)

        pltpu.sync_copy(acc_v, scorep_hbm.at[w])

    return k(stilde, src23, dst3, zeros1d)


def _tc_prep(x, W, degp):
    din = x.shape[1]

    def body(x_ref, w_ref, degp_ref, z_ref, xwd2_ref, dinv_ref):
        deg = jnp.sum(degp_ref[...], axis=0) + 1.0
        dinv = lax.rsqrt(deg)
        xw = jnp.dot(x_ref[...], w_ref[...],
                     preferred_element_type=jnp.float32)
        dc = dinv[:, None]
        z_ref[...] = xw * dc
        xwd2_ref[...] = xw * (dc * dc)
        dinv_ref[...] = dinv[None, None, :]

    return pl.pallas_call(
        body,
        grid=(NB,),
        in_specs=[pl.BlockSpec((128, din), lambda i: (i, 0)),
                  pl.BlockSpec((din, H), lambda i: (0, 0)),
                  pl.BlockSpec((NW, 128), lambda i: (0, i))],
        out_specs=[pl.BlockSpec((128, H), lambda i: (i, 0)),
                   pl.BlockSpec((128, H), lambda i: (i, 0)),
                   pl.BlockSpec((1, 1, 128), lambda i: (i, 0, 0))],
        out_shape=[jax.ShapeDtypeStruct((NT, H), jnp.float32),
                   jax.ShapeDtypeStruct((NT, H), jnp.float32),
                   jax.ShapeDtypeStruct((NB, 1, 128), jnp.float32)],
    )(x, W, degp)


def _tc_post_agg(p, xwd2, dinv2, b, wsT, bs):
    def body(p_ref, xwd2_ref, dinv_ref, b_ref, ws_ref, bs_ref,
             y_ref, st_ref, ss_ref):
        i = pl.program_id(0)
        dinv = dinv_ref[0, 0, :]
        dc = dinv[:, None]
        es = p_ref[0] + p_ref[1]
        y = jnp.maximum(es * dc + xwd2_ref[...] + b_ref[...], 0.0)
        rowid = i * 128 + lax.broadcasted_iota(jnp.int32, (128, 1), 0)
        y = jnp.where(rowid < N_REAL, y, 0.0)
        y_ref[...] = y
        shat = jnp.sum(y * ws_ref[...], axis=1)
        st_ref[...] = (shat * dinv)[None, None, :]
        ss_ref[...] = (shat * dinv * dinv + bs_ref[0, 0])[None, None, :]

    return pl.pallas_call(
        body,
        grid=(NB,),
        in_specs=[pl.BlockSpec((2, 128, H), lambda i: (0, i, 0)),
                  pl.BlockSpec((128, H), lambda i: (i, 0)),
                  pl.BlockSpec((1, 1, 128), lambda i: (i, 0, 0)),
                  pl.BlockSpec((1, H), lambda i: (0, 0)),
                  pl.BlockSpec((1, H), lambda i: (0, 0)),
                  pl.BlockSpec((1, 1), lambda i: (0, 0))],
        out_specs=[pl.BlockSpec((128, H), lambda i: (i, 0)),
                   pl.BlockSpec((1, 1, 128), lambda i: (i, 0, 0)),
                   pl.BlockSpec((1, 1, 128), lambda i: (i, 0, 0))],
        out_shape=[jax.ShapeDtypeStruct((NT, H), jnp.float32),
                   jax.ShapeDtypeStruct((NB, 1, 128), jnp.float32),
                   jax.ShapeDtypeStruct((NB, 1, 128), jnp.float32)],
    )(p, xwd2, dinv2, b, wsT, bs)


def _tc_topk(scorep3, dinv2, ss2, alive2, k):
    def body(sp_ref, dinv_ref, ss_ref, alive_ref, scale_ref, anew_ref):
        es = jnp.sum(sp_ref[...], axis=0)
        score = dinv_ref[...] * es + ss_ref[...]
        u = lax.bitcast_convert_type(score, jnp.uint32)
        flip = jnp.where(u >= jnp.uint32(0x80000000),
                         jnp.uint32(0xFFFFFFFF), jnp.uint32(0x80000000))
        key = jnp.where(alive_ref[...] > 0.0, u ^ flip, jnp.uint32(0))

        def step(i, t):
            bit = jnp.uint32(31) - i.astype(jnp.uint32)
            cand = t | (jnp.uint32(1) << bit)
            cnt = jnp.sum((key >= cand).astype(jnp.int32))
            return jnp.where(cnt >= k, cand, t)

        thr = lax.fori_loop(0, 32, step, jnp.uint32(0))
        anew = (key >= thr).astype(jnp.float32)
        anew_ref[...] = anew
        scale_ref[...] = jnp.tanh(score) * anew

    return pl.pallas_call(
        body,
        in_specs=[pl.BlockSpec((NW, NB, 128), lambda: (0, 0, 0)),
                  pl.BlockSpec((NB, 128), lambda: (0, 0)),
                  pl.BlockSpec((NB, 128), lambda: (0, 0)),
                  pl.BlockSpec((NB, 128), lambda: (0, 0))],
        out_specs=[pl.BlockSpec((NB, 128), lambda: (0, 0)),
                   pl.BlockSpec((NB, 128), lambda: (0, 0))],
        out_shape=[jax.ShapeDtypeStruct((NB, 128), jnp.float32),
                   jax.ShapeDtypeStruct((NB, 128), jnp.float32)],
    )(scorep3, dinv2, ss2, alive2)


def _tc_pool(y, scale2, alive2):
    def body(y_ref, sc_ref, a_ref, xn_ref, rp_ref):
        s = sc_ref[0, 0, :][:, None]
        a = a_ref[0, 0, :][:, None]
        xn = y_ref[...] * s
        xn_ref[...] = xn
        mx = jnp.max(jnp.where(a > 0.0, xn, -jnp.inf), axis=0)
        sm = jnp.sum(xn, axis=0)
        rp_ref[...] = jnp.concatenate([mx, sm])[None, None, :]

    return pl.pallas_call(
        body,
        grid=(NB,),
        in_specs=[pl.BlockSpec((128, H), lambda i: (i, 0)),
                  pl.BlockSpec((1, 1, 128), lambda i: (i, 0, 0)),
                  pl.BlockSpec((1, 1, 128), lambda i: (i, 0, 0))],
        out_specs=[pl.BlockSpec((128, H), lambda i: (i, 0)),
                   pl.BlockSpec((1, 1, 128), lambda i: (i, 0, 0))],
        out_shape=[jax.ShapeDtypeStruct((NT, H), jnp.float32),
                   jax.ShapeDtypeStruct((NB, 1, 128), jnp.float32)],
    )(y, scale2, alive2)


def _tc_final(rp1, rp2, rp3, w1, b1, w2, b2, w3, b3):
    def body(r1, r2, r3, w1r, b1r, w2r, b2r, w3r, b3r, hn_ref, hc_ref):
        h = jnp.zeros((1, 128), jnp.float32)
        for rp, k in ((r1, KS[0]), (r2, KS[1]), (r3, KS[2])):
            v = rp[...]
            mx = jnp.max(v[:, :H], axis=0)
            sm = jnp.sum(v[:, H:], axis=0) * (1.0 / k)
            h = h + jnp.concatenate([mx, sm])[None, :]
        h = jnp.maximum(jnp.dot(h, w1r[...],
                                preferred_element_type=jnp.float32)
                        + b1r[...], 0.0)
        h = jnp.maximum(jnp.dot(h, w2r[...],
                                preferred_element_type=jnp.float32)
                        + b2r[...], 0.0)
        nrm = jnp.maximum(jnp.sqrt(jnp.sum(h * h)), 1e-12)
        hn_ref[...] = h / nrm
        hc_ref[...] = jnp.dot(h, w3r[...],
                              preferred_element_type=jnp.float32) + b3r[...]

    def full(s):
        return pl.BlockSpec(s, lambda *_: tuple(0 for _ in s))

    return pl.pallas_call(
        body,
        in_specs=[full((NB, 128)), full((NB, 128)), full((NB, 128)),
                  full((2 * H, H)), full((1, H)),
                  full((H, 32)), full((1, 32)),
                  full((32, 2)), full((1, 2))],
        out_specs=[full((1, 32)), full((1, 2))],
        out_shape=[jax.ShapeDtypeStruct((1, 32), jnp.float32),
                   jax.ShapeDtypeStruct((1, 2), jnp.float32)],
    )(rp1, rp2, rp3, w1, b1, w2, b2, w3, b3)


def _tower(x, src, dst, stages, head, consts):
    zeros1d, zeros2d, alive0 = consts
    xp = jnp.pad(x, ((0, NT - N_REAL), (0, 0)))
    pad_n = EPAD - E
    ar = jnp.arange(pad_n, dtype=jnp.int32)
    src3 = jnp.concatenate([src, ZBASE + (ar % 256)]).reshape(NW, CH, 128)
    dst3 = jnp.concatenate([dst, ar % N_REAL]).reshape(NW, CH, 128)

    alive = alive0
    xcur = xp
    rps = []
    for (W, b, wsT, bs), k in zip(stages, KS):
        degp, src2 = _sc_pass_a(alive.reshape(NT), src3, dst3, zeros1d)
        z, xwd2, dinv3 = _tc_prep(xcur, W, degp)
        p = _sc_pass_b(z, src2, dst3, zeros2d)
        y, st3, ss3 = _tc_post_agg(p, xwd2, dinv3, b, wsT, bs)
        scorep = _sc_pass_c(st3.reshape(NT), src2, dst3, zeros1d)
        scale2, anew2 = _tc_topk(scorep.reshape(NW, NB, 128),
                                 dinv3.reshape(NB, 128),
                                 ss3.reshape(NB, 128), alive, k)
        xcur, rp = _tc_pool(y, scale2.reshape(NB, 1, 128),
                            anew2.reshape(NB, 1, 128))
        alive = anew2
        rps.append(rp)
    return _tc_final(rps[0].reshape(NB, 128), rps[1].reshape(NB, 128),
                     rps[2].reshape(NB, 128), *head)


def kernel(x1, x2, edge_index1, edge_index2, batch1, batch2,
           W1, b1, Ws1, bs1, W2, b2, Ws2, bs2, W3, b3, Ws3, bs3,
           Wl1, bl1, Wl2, bl2, Wl3, bl3):
    zeros1d = jnp.zeros((NT,), jnp.float32)
    zeros2d = jnp.zeros((NT, H), jnp.float32)
    alive0 = (jnp.arange(NT) < N_REAL).astype(jnp.float32).reshape(NB, 128)
    consts = (zeros1d, zeros2d, alive0)

    stages = [(W1, b1.reshape(1, H), Ws1.reshape(1, H), bs1.reshape(1, 1)),
              (W2, b2.reshape(1, H), Ws2.reshape(1, H), bs2.reshape(1, 1)),
              (W3, b3.reshape(1, H), Ws3.reshape(1, H), bs3.reshape(1, 1))]
    head = (Wl1, bl1.reshape(1, H), Wl2, bl2.reshape(1, 32),
            Wl3, bl3.reshape(1, 2))

    n1, c1 = _tower(x1, edge_index1[0], edge_index1[1], stages, head, consts)
    n2, c2 = _tower(x2, edge_index2[0], edge_index2[1], stages, head, consts)
    return (n1, c1, n2, c2)
